# baseline (speedup 1.0000x reference)
import functools

import jax
import jax.numpy as jnp
from jax import lax
from jax.experimental import pallas as pl
from jax.experimental.pallas import tpu as pltpu
from jax.experimental.pallas import tpu_sc as plsc

BATCH = 16384
EMBED = 32
FEAT = 2 * EMBED
NW = 32
CHUNK = 128
B_PER_W = BATCH // NW
NCH = B_PER_W // CHUNK
BN_EPS = 1e-5

_mesh = plsc.VectorSubcoreMesh(core_axis_name="c", subcore_axis_name="s")


@functools.partial(
    pl.kernel,
    mesh=_mesh,
    compiler_params=pltpu.CompilerParams(use_tc_tiling_on_sc=False),
    out_type=jax.ShapeDtypeStruct((NW, NCH, CHUNK, 2, EMBED), jnp.float32),
    scratch_types=[
        pltpu.VMEM((NCH, CHUNK), jnp.int32),
        pltpu.VMEM((NCH, CHUNK), jnp.int32),
        pltpu.VMEM((NCH, CHUNK, EMBED), jnp.float32),
        pltpu.VMEM((NCH, CHUNK, EMBED), jnp.float32),
        pltpu.SemaphoreType.DMA,
    ],
)
def _gather_embeddings(users_hbm, items_hbm, utab_hbm, mtab_hbm, out_hbm,
                       uidx_v, iidx_v, ubuf_v, ibuf_v, sem):
    c = lax.axis_index("c")
    s = lax.axis_index("s")
    wid = s * 2 + c
    pltpu.sync_copy(users_hbm.at[wid], uidx_v)
    pltpu.sync_copy(items_hbm.at[wid], iidx_v)
    copies = []
    for j in range(NCH):
        copies.append(pltpu.async_copy(utab_hbm.at[uidx_v.at[j]], ubuf_v.at[j], sem))
        copies.append(pltpu.async_copy(mtab_hbm.at[iidx_v.at[j]], ibuf_v.at[j], sem))
    for cp in copies:
        cp.wait()
    for j in range(NCH):
        pltpu.sync_copy(ubuf_v.at[j], out_hbm.at[wid, j, :, 0])
        pltpu.sync_copy(ibuf_v.at[j], out_hbm.at[wid, j, :, 1])


BM = 2048


def _mlp_body(x_ref, g_ref, be_ref, mu_ref, var_ref,
              W1_ref, b1_ref, W2_ref, b2_ref, W3_ref, b3_ref, Wo_ref, bo_ref,
              o_ref):
    inv = lax.rsqrt(var_ref[...] + BN_EPS)
    x = (x_ref[...] - mu_ref[...]) * (g_ref[...] * inv) + be_ref[...]
    h = jnp.maximum(jnp.dot(x, W1_ref[...], preferred_element_type=jnp.float32) + b1_ref[...], 0.0)
    h = jnp.maximum(jnp.dot(h, W2_ref[...], preferred_element_type=jnp.float32) + b2_ref[...], 0.0)
    h = jnp.maximum(jnp.dot(h, W3_ref[...], preferred_element_type=jnp.float32) + b3_ref[...], 0.0)
    o_ref[...] = jnp.dot(h, Wo_ref[...], preferred_element_type=jnp.float32) + bo_ref[...]


def _full(shape):
    return pl.BlockSpec(shape, lambda i: (0, 0))


_mlp = pl.pallas_call(
    _mlp_body,
    grid=(BATCH // BM,),
    in_specs=[
        pl.BlockSpec((BM, FEAT), lambda i: (i, 0)),
        _full((1, FEAT)), _full((1, FEAT)), _full((1, FEAT)), _full((1, FEAT)),
        _full((FEAT, 32)), _full((1, 32)),
        _full((32, 16)), _full((1, 16)),
        _full((16, 8)), _full((1, 8)),
        _full((8, 1)), _full((1, 1)),
    ],
    out_specs=pl.BlockSpec((BM, 1), lambda i: (i, 0)),
    out_shape=jax.ShapeDtypeStruct((BATCH, 1), jnp.float32),
)


def kernel(users, items, user_table, movie_table, bn_gamma, bn_beta, bn_mean,
           bn_var, W1, b1, W2, b2, W3, b3, Wo, bo):
    users_r = users.astype(jnp.int32).reshape(NW, NCH, CHUNK)
    items_r = items.astype(jnp.int32).reshape(NW, NCH, CHUNK)
    x = _gather_embeddings(users_r, items_r, user_table, movie_table)
    x = x.reshape(BATCH, FEAT)
    rating = _mlp(
        x,
        bn_gamma.reshape(1, FEAT), bn_beta.reshape(1, FEAT),
        bn_mean.reshape(1, FEAT), bn_var.reshape(1, FEAT),
        W1, b1.reshape(1, 32),
        W2, b2.reshape(1, 16),
        W3, b3.reshape(1, 8),
        Wo, bo.reshape(1, 1),
    )
    return rating
